# baseline (speedup 1.0000x reference)
import functools

import jax
import jax.numpy as jnp
from jax import lax
from jax.experimental import pallas as pl

B = 16
H = 16
C = 128
W = 512
OV = 4
NVQ = 8
CD = 32
K = 1024
FIX = H * C
INVQ = FIX * OV // NVQ
T = W // OV
M = B * T
MT = 256
HIGH = lax.Precision.DEFAULT


def _vq_body(z_ref, wd_ref, wu_ref, cb_ref, zq_ref, zn_ref, code_ref, cm_ref):
    zg = z_ref[...]
    wd = wd_ref[0]
    zd = lax.dot_general(zg, wd, (((1,), (1,)), ((), ())), precision=HIGH)
    nrm = jnp.sqrt(jnp.sum(zd * zd, axis=-1, keepdims=True))
    zn = zd / (nrm + 1e-8)

    emb = cb_ref[0]
    enrm = jnp.sqrt(jnp.sum(emb * emb, axis=-1, keepdims=True))
    en = emb / (enrm + 1e-8)
    ensq = jnp.sum(en * en, axis=-1)
    znsq = jnp.sum(zn * zn, axis=-1, keepdims=True)

    dots = lax.dot_general(zn, en, (((1,), (1,)), ((), ())), precision=HIGH)
    d = znsq - 2.0 * dots + ensq[None, :]
    dmin = jnp.min(d, axis=-1, keepdims=True)
    iota = lax.broadcasted_iota(jnp.int32, (MT, K), 1)
    code = jnp.min(jnp.where(d == dmin, iota, K), axis=-1)
    oh = (iota == code[:, None]).astype(jnp.float32)
    zq_down = lax.dot_general(oh, en, (((1,), (0,)), ((), ())), precision=HIGH)

    diff = zn - zq_down
    cm_part = jnp.sum(diff * diff)

    wu = wu_ref[0]
    zq = lax.dot_general(zq_down, wu, (((1,), (1,)), ((), ())), precision=HIGH)

    zq_ref[...] = zq
    zn_ref[0] = zn
    code_ref[0, 0, :] = code

    @pl.when((pl.program_id(0) == 0) & (pl.program_id(1) == 0))
    def _():
        cm_ref[...] = jnp.zeros((1, 1), jnp.float32)

    cm_ref[...] += jnp.reshape(cm_part, (1, 1))


@functools.partial(jax.jit)
def _vq_core(z2, W_down, W_up, codebooks):
    grid = (NVQ, M // MT)
    out_shapes = (
        jax.ShapeDtypeStruct((M, NVQ * INVQ), jnp.float32),
        jax.ShapeDtypeStruct((NVQ, M, CD), jnp.float32),
        jax.ShapeDtypeStruct((NVQ, 1, M), jnp.int32),
        jax.ShapeDtypeStruct((1, 1), jnp.float32),
    )
    return pl.pallas_call(
        _vq_body,
        grid=grid,
        in_specs=[
            pl.BlockSpec((MT, INVQ), lambda g, m: (m, g)),
            pl.BlockSpec((1, CD, INVQ), lambda g, m: (g, 0, 0)),
            pl.BlockSpec((1, INVQ, CD), lambda g, m: (g, 0, 0)),
            pl.BlockSpec((1, K, CD), lambda g, m: (g, 0, 0)),
        ],
        out_specs=(
            pl.BlockSpec((MT, INVQ), lambda g, m: (m, g)),
            pl.BlockSpec((1, MT, CD), lambda g, m: (g, m, 0)),
            pl.BlockSpec((1, 1, MT), lambda g, m: (g, 0, m)),
            pl.BlockSpec((1, 1), lambda g, m: (0, 0)),
        ),
        out_shape=out_shapes,
    )(z2, W_down, W_up, codebooks)


def kernel(z_e, W_down, W_up, codebooks):
    z = z_e.reshape(B, H, W, C).transpose(0, 2, 3, 1).reshape(B, W, FIX)
    z2 = z.reshape(M, NVQ * INVQ)
    zq_all, zn_all, codes, cmsum = _vq_core(z2, W_down, W_up, codebooks)

    zq = (zq_all.reshape(B, W, C, H)
          .transpose(0, 3, 1, 2).reshape(B, H * W, C))
    z_e_downs = zn_all.reshape(NVQ, B, T, CD).transpose(1, 0, 2, 3)
    indices = codes.reshape(NVQ, B, T).transpose(1, 0, 2)
    cm = cmsum[0, 0] / (NVQ * M * CD)
    return (zq, z_e_downs, indices, cm, cm)


# speedup vs baseline: 3.7960x; 3.7960x over previous
import functools

import jax
import jax.numpy as jnp
from jax import lax
from jax.experimental import pallas as pl

B = 16
H = 16
C = 128
W = 512
OV = 4
NVQ = 8
CD = 32
K = 1024
FIX = H * C
INVQ = FIX * OV // NVQ
T = W // OV
GD = NVQ * CD
DEF = lax.Precision.DEFAULT
HIGH = lax.Precision.HIGHEST


def _vq_body(x_ref, wdbig_ref, wubig_ref, cb_ref, psel_ref,
             zq_ref, zn_ref, code_ref, cm_ref):
    x = x_ref[0]
    psel = psel_ref[...]

    s = lax.dot_general(x[0], wdbig_ref[0], (((1,), (0,)), ((), ())),
                        precision=DEF)
    for h in range(1, H):
        s = s + lax.dot_general(x[h], wdbig_ref[h], (((1,), (0,)), ((), ())),
                                precision=DEF)

    zsel = lax.dot_general(psel, s, (((1,), (0,)), ((), ())),
                           precision=HIGH)

    zn_list, code_list, q_list = [], [], []
    cm_part = jnp.zeros((), jnp.float32)
    iota = lax.broadcasted_iota(jnp.int32, (T, K), 1)
    for g in range(NVQ):
        ov = g // 2
        zd = zsel[ov * T:(ov + 1) * T, g * CD:(g + 1) * CD]
        nrm = jnp.sqrt(jnp.sum(zd * zd, axis=-1, keepdims=True))
        zn = zd / (nrm + 1e-8)

        emb = cb_ref[g]
        enrm = jnp.sqrt(jnp.sum(emb * emb, axis=-1, keepdims=True))
        en = emb / (enrm + 1e-8)
        ensq = jnp.sum(en * en, axis=-1)
        znsq = jnp.sum(zn * zn, axis=-1, keepdims=True)

        dots = lax.dot_general(zn, en, (((1,), (1,)), ((), ())), precision=DEF)
        d = znsq - 2.0 * dots + ensq[None, :]
        dmin = jnp.min(d, axis=-1, keepdims=True)
        code = jnp.min(jnp.where(d == dmin, iota, K), axis=-1)
        oh = (iota == code[:, None]).astype(jnp.float32)
        zq_down = lax.dot_general(oh, en, (((1,), (0,)), ((), ())),
                                  precision=DEF)

        diff = zn - zq_down
        cm_part = cm_part + jnp.sum(diff * diff)
        zn_list.append(zn)
        code_list.append(code)
        q_list.append(zq_down)

    zero = jnp.zeros((T, CD), jnp.float32)
    rows = []
    for ov in range(OV):
        pieces = [q_list[g] if g // 2 == ov else zero for g in range(NVQ)]
        rows.append(jnp.concatenate(pieces, axis=1))
    qstack = jnp.concatenate(rows, axis=0)

    q = lax.dot_general(psel, qstack, (((0,), (0,)), ((), ())),
                        precision=HIGH)

    for h in range(H):
        zq_ref[0, h] = lax.dot_general(q, wubig_ref[h], (((1,), (0,)), ((), ())),
                                       precision=DEF)

    zn_ref[0] = jnp.stack(zn_list, axis=0)
    code_ref[0] = jnp.stack(code_list, axis=0)

    @pl.when(pl.program_id(0) == 0)
    def _():
        cm_ref[...] = jnp.zeros((1, 1), jnp.float32)

    cm_ref[...] += jnp.reshape(cm_part, (1, 1))


@functools.partial(jax.jit)
def _vq_core(z4, wdbig, wubig, codebooks, psel):
    out_shapes = (
        jax.ShapeDtypeStruct((B, H, W, C), jnp.float32),
        jax.ShapeDtypeStruct((B, NVQ, T, CD), jnp.float32),
        jax.ShapeDtypeStruct((B, NVQ, T), jnp.int32),
        jax.ShapeDtypeStruct((1, 1), jnp.float32),
    )
    return pl.pallas_call(
        _vq_body,
        grid=(B,),
        in_specs=[
            pl.BlockSpec((1, H, W, C), lambda b: (b, 0, 0, 0)),
            pl.BlockSpec((H, C, GD), lambda b: (0, 0, 0)),
            pl.BlockSpec((H, GD, C), lambda b: (0, 0, 0)),
            pl.BlockSpec((NVQ, K, CD), lambda b: (0, 0, 0)),
            pl.BlockSpec((W, W), lambda b: (0, 0)),
        ],
        out_specs=(
            pl.BlockSpec((1, H, W, C), lambda b: (b, 0, 0, 0)),
            pl.BlockSpec((1, NVQ, T, CD), lambda b: (b, 0, 0, 0)),
            pl.BlockSpec((1, NVQ, T), lambda b: (b, 0, 0)),
            pl.BlockSpec((1, 1), lambda b: (0, 0)),
        ),
        out_shape=out_shapes,
    )(z4, wdbig, wubig, codebooks, psel)


def _prep_weights(W_down, W_up):
    wd4 = W_down.reshape(NVQ, CD, 64, H).transpose(3, 0, 2, 1)
    p = wd4.transpose(0, 2, 1, 3).reshape(H, 64, GD)
    m = jnp.repeat((jnp.arange(NVQ) % 2 == 0), CD).astype(jnp.float32)
    wdbig = jnp.concatenate([p * m[None, None, :],
                             p * (1.0 - m)[None, None, :]], axis=1)

    wu4 = W_up.reshape(NVQ, 64, H, CD).transpose(2, 0, 3, 1)
    quu = wu4.reshape(H, GD, 64)
    wubig = jnp.concatenate([quu * m[None, :, None],
                             quu * (1.0 - m)[None, :, None]], axis=2)
    return wdbig, wubig


def kernel(z_e, W_down, W_up, codebooks):
    z4 = z_e.reshape(B, H, W, C)
    wdbig, wubig = _prep_weights(W_down, W_up)
    tt = jnp.arange(W) // OV
    ovv = jnp.arange(W) % OV
    psel = jnp.zeros((W, W), jnp.float32).at[ovv * T + tt, jnp.arange(W)].set(1.0)
    zq4, zn_out, codes, cmsum = _vq_core(z4, wdbig, wubig, codebooks, psel)
    zq = zq4.reshape(B, H * W, C)
    cm = cmsum[0, 0] / (NVQ * B * T * CD)
    return (zq, zn_out, codes, cm, cm)
